# Initial kernel scaffold; baseline (speedup 1.0000x reference)
import jax
import jax.numpy as jnp
from jax.experimental import pallas as pl


def kernel(z, embedding):
    raise NotImplementedError("write your pallas kernel here")


# speedup vs baseline: 1.5096x; 1.5096x over previous
import functools

import jax
import jax.numpy as jnp
from jax import lax
from jax.experimental import pallas as pl
from jax.experimental.pallas import tpu as pltpu
from jax.experimental.pallas import tpu_sc as plsc

D = 64
K = 1024
ROWS = 512


def _tc_body(nblocks, z_ref, emb_ref, idx_ref, en_ref, loss_ref,
             esq_ref, acc_ref):
    i = pl.program_id(0)

    @pl.when(i == 0)
    def _init():
        e = emb_ref[...]
        n = jnp.sqrt(jnp.sum(e * e, axis=1, keepdims=True))
        en = e / jnp.maximum(n, 1e-12)
        en_ref[...] = en
        esq_ref[...] = lax.dot_general(
            jnp.ones((1, D), jnp.float32), en * en,
            (((1,), (1,)), ((), ())), precision=lax.Precision.HIGHEST,
            preferred_element_type=jnp.float32)
        acc_ref[...] = jnp.zeros((1, 1), jnp.float32)

    z = z_ref[...]
    n = jnp.sqrt(jnp.sum(z * z, axis=1, keepdims=True))
    zn = z / jnp.maximum(n, 1e-12)
    znsq = jnp.sum(zn * zn, axis=1, keepdims=True)
    dot = lax.dot_general(zn, en_ref[...], (((1,), (1,)), ((), ())),
                          preferred_element_type=jnp.float32)
    d = (znsq - 2.0 * dot) + esq_ref[...]
    dmin = jnp.min(d, axis=1, keepdims=True)
    ji = lax.broadcasted_iota(jnp.int32, d.shape, 1)
    idx_ref[...] = jnp.min(jnp.where(d == dmin, ji, K), axis=1,
                           keepdims=True)
    acc_ref[...] += jnp.sum(dmin, axis=0, keepdims=True)

    @pl.when(i == nblocks - 1)
    def _fin():
        nrows = nblocks * ROWS
        loss_ref[...] = acc_ref[...] * (2.0 / (nrows * D))


def _tc_quantize(zf, embedding):
    n = zf.shape[0]
    nblocks = n // ROWS
    return pl.pallas_call(
        functools.partial(_tc_body, nblocks),
        grid=(nblocks,),
        in_specs=[
            pl.BlockSpec((ROWS, D), lambda i: (i, 0)),
            pl.BlockSpec((K, D), lambda i: (0, 0)),
        ],
        out_specs=[
            pl.BlockSpec((ROWS, 1), lambda i: (i, 0)),
            pl.BlockSpec((K, D), lambda i: (0, 0)),
            pl.BlockSpec((1, 1), lambda i: (0, 0)),
        ],
        out_shape=[
            jax.ShapeDtypeStruct((n, 1), jnp.int32),
            jax.ShapeDtypeStruct((K, D), jnp.float32),
            jax.ShapeDtypeStruct((1, 1), jnp.float32),
        ],
        scratch_shapes=[
            pltpu.VMEM((1, K), jnp.float32),
            pltpu.VMEM((1, 1), jnp.float32),
        ],
    )(zf, embedding)


_NC = 2
_NS = 16
_NW = _NC * _NS
_CHUNK = 128


def _sc_gather(en, idx3):
    ch = idx3.shape[1]
    mesh = plsc.VectorSubcoreMesh(core_axis_name="c", subcore_axis_name="s")

    @functools.partial(
        pl.kernel, mesh=mesh,
        compiler_params=pltpu.CompilerParams(use_tc_tiling_on_sc=False),
        out_type=jax.ShapeDtypeStruct((_NW, ch, _CHUNK, D), jnp.float32),
        scratch_types=[
            pltpu.VMEM((ch, _CHUNK), jnp.int32),
            pltpu.VMEM((ch, _CHUNK, D), jnp.float32),
            pltpu.SemaphoreType.DMA,
        ],
    )
    def gather(en_hbm, idx_hbm, out_hbm, idx_v, rows_v, sem):
        wid = lax.axis_index("s") * _NC + lax.axis_index("c")
        pltpu.sync_copy(idx_hbm.at[wid], idx_v)
        copies = [
            pltpu.async_copy(en_hbm.at[idx_v.at[j]], rows_v.at[j], sem)
            for j in range(ch)
        ]
        for c in copies:
            c.wait()
        pltpu.sync_copy(rows_v, out_hbm.at[wid])

    return gather(en, idx3)


def kernel(z, embedding):
    b, hw, d = z.shape
    zf = z.reshape(-1, d)
    idx_col, en, loss11 = _tc_quantize(zf, embedding)
    idx3 = idx_col.reshape(_NW, -1, _CHUNK)
    zq = _sc_gather(en, idx3).reshape(z.shape)
    loss = loss11.reshape(())
    return (zq, loss, idx_col.reshape(b, hw))


# speedup vs baseline: 1.5154x; 1.0038x over previous
import functools

import jax
import jax.numpy as jnp
from jax import lax
from jax.experimental import pallas as pl
from jax.experimental.pallas import tpu as pltpu
from jax.experimental.pallas import tpu_sc as plsc

D = 64
K = 1024
ROWS = 512


def _tc_body(nblocks, z_ref, emb_ref, idx_ref, en_ref, loss_ref,
             esq_ref, acc_ref):
    i = pl.program_id(0)

    @pl.when(i == 0)
    def _init():
        e = emb_ref[...]
        n = jnp.sqrt(jnp.sum(e * e, axis=1, keepdims=True))
        en = e / jnp.maximum(n, 1e-12)
        en_ref[...] = en
        esq_ref[...] = 0.5 * lax.dot_general(
            jnp.ones((1, D), jnp.float32), en * en,
            (((1,), (1,)), ((), ())), precision=lax.Precision.HIGHEST,
            preferred_element_type=jnp.float32)
        acc_ref[...] = jnp.zeros((1, 1), jnp.float32)

    z = z_ref[...]
    n = jnp.sqrt(jnp.sum(z * z, axis=1, keepdims=True))
    zn = z / jnp.maximum(n, 1e-12)
    znsq = jnp.sum(zn * zn, axis=1, keepdims=True)
    dot = lax.dot_general(zn, en_ref[...], (((1,), (1,)), ((), ())),
                          preferred_element_type=jnp.float32)
    s = dot - esq_ref[...]
    mx = jnp.max(s, axis=1, keepdims=True)
    ji = lax.broadcasted_iota(jnp.int32, s.shape, 1)
    idx_ref[...] = jnp.min(jnp.where(s == mx, ji, K), axis=1,
                           keepdims=True)
    acc_ref[...] += (jnp.sum(znsq, axis=0, keepdims=True)
                     - 2.0 * jnp.sum(mx, axis=0, keepdims=True))

    @pl.when(i == nblocks - 1)
    def _fin():
        nrows = nblocks * ROWS
        loss_ref[...] = acc_ref[...] * (2.0 / (nrows * D))


def _tc_quantize(zf, embedding):
    n = zf.shape[0]
    nblocks = n // ROWS
    return pl.pallas_call(
        functools.partial(_tc_body, nblocks),
        grid=(nblocks,),
        in_specs=[
            pl.BlockSpec((ROWS, D), lambda i: (i, 0)),
            pl.BlockSpec((K, D), lambda i: (0, 0)),
        ],
        out_specs=[
            pl.BlockSpec((ROWS, 1), lambda i: (i, 0)),
            pl.BlockSpec((K, D), lambda i: (0, 0)),
            pl.BlockSpec((1, 1), lambda i: (0, 0)),
        ],
        out_shape=[
            jax.ShapeDtypeStruct((n, 1), jnp.int32),
            jax.ShapeDtypeStruct((K, D), jnp.float32),
            jax.ShapeDtypeStruct((1, 1), jnp.float32),
        ],
        scratch_shapes=[
            pltpu.VMEM((1, K), jnp.float32),
            pltpu.VMEM((1, 1), jnp.float32),
        ],
    )(zf, embedding)


_NC = 2
_NS = 16
_NW = _NC * _NS
_CHUNK = 128


def _sc_gather(en, idx3):
    ch = idx3.shape[1]
    mesh = plsc.VectorSubcoreMesh(core_axis_name="c", subcore_axis_name="s")

    @functools.partial(
        pl.kernel, mesh=mesh,
        compiler_params=pltpu.CompilerParams(use_tc_tiling_on_sc=False),
        out_type=jax.ShapeDtypeStruct((_NW, ch, _CHUNK, D), jnp.float32),
        scratch_types=[
            pltpu.VMEM((ch, _CHUNK), jnp.int32),
            pltpu.VMEM((ch, _CHUNK, D), jnp.float32),
            pltpu.SemaphoreType.DMA,
        ],
    )
    def gather(en_hbm, idx_hbm, out_hbm, idx_v, rows_v, sem):
        wid = lax.axis_index("s") * _NC + lax.axis_index("c")
        pltpu.sync_copy(idx_hbm.at[wid], idx_v)
        copies = [
            pltpu.async_copy(en_hbm.at[idx_v.at[j]], rows_v.at[j], sem)
            for j in range(ch)
        ]
        for c in copies:
            c.wait()
        pltpu.sync_copy(rows_v, out_hbm.at[wid])

    return gather(en, idx3)


def kernel(z, embedding):
    b, hw, d = z.shape
    zf = z.reshape(-1, d)
    idx_col, en, loss11 = _tc_quantize(zf, embedding)
    idx3 = idx_col.reshape(_NW, -1, _CHUNK)
    zq = _sc_gather(en, idx3).reshape(z.shape)
    loss = loss11.reshape(())
    return (zq, loss, idx_col.reshape(b, hw))
